# Initial kernel scaffold; baseline (speedup 1.0000x reference)
import jax
import jax.numpy as jnp
from jax.experimental import pallas as pl


def kernel(x, pos_embedding):
    raise NotImplementedError("write your pallas kernel here")


# speedup vs baseline: 1.7204x; 1.7204x over previous
import jax
import jax.numpy as jnp
from jax.experimental import pallas as pl

_BT = 256


def _add_kernel(x_ref, pos_ref, o_ref):
    o_ref[...] = x_ref[...] + pos_ref[...][None, :, :]


def kernel(x, pos_embedding):
    B, T, E = x.shape
    grid = (T // _BT,)
    return pl.pallas_call(
        _add_kernel,
        grid=grid,
        in_specs=[
            pl.BlockSpec((B, _BT, E), lambda t: (0, t, 0)),
            pl.BlockSpec((_BT, E), lambda t: (t, 0)),
        ],
        out_specs=pl.BlockSpec((B, _BT, E), lambda t: (0, t, 0)),
        out_shape=jax.ShapeDtypeStruct((B, T, E), x.dtype),
    )(x, pos_embedding)


# speedup vs baseline: 1.7230x; 1.0015x over previous
import jax
import jax.numpy as jnp
from jax.experimental import pallas as pl

_BT = 512


def _add_kernel(x_ref, pos_ref, o_ref):
    o_ref[...] = x_ref[...] + pos_ref[...][None, :, :]


def kernel(x, pos_embedding):
    B, T, E = x.shape
    grid = (T // _BT,)
    return pl.pallas_call(
        _add_kernel,
        grid=grid,
        in_specs=[
            pl.BlockSpec((B, _BT, E), lambda t: (0, t, 0)),
            pl.BlockSpec((_BT, E), lambda t: (t, 0)),
        ],
        out_specs=pl.BlockSpec((B, _BT, E), lambda t: (0, t, 0)),
        out_shape=jax.ShapeDtypeStruct((B, T, E), x.dtype),
    )(x, pos_embedding)
